# baseline (speedup 1.0000x reference)
import jax
import jax.numpy as jnp
from jax.experimental import pallas as pl
from jax.experimental.pallas import tpu as pltpu

_STATE_DIM = 128
_SEM_DIM = 64
_SLOTS = 8192
_LR = 0.01
_N = 4096
_LANE = 128
_MAC = 16
_MACSZ = _LANE * _MAC
_INF = float('inf')


def _consolidate_kernel(states_ref, rewards_ref, w_ref, b_ref,
                        traces_ref, num_ref, ms_ref,
                        sem_ref, strengths_ref, tt_ref):
    sem_ref[...] = jax.lax.dot_general(
        states_ref[...], w_ref[...],
        dimension_numbers=(((1,), (1,)), ((), ())),
        preferred_element_type=jnp.float32) + b_ref[...]

    strengths_ref[...] = jnp.zeros((_SLOTS, 1), jnp.float32)
    tt_ref[...] = jnp.zeros((_SLOTS // _LANE, _SEM_DIM, _LANE), jnp.float32)

    row_ids = jax.lax.broadcasted_iota(jnp.int32, (_SLOTS, 1), 0)
    seg_ids = jax.lax.broadcasted_iota(jnp.int32, (_MAC, _LANE), 0)
    lane_ids = jax.lax.broadcasted_iota(jnp.int32, (_MAC, _LANE), 1)
    lane_ids3 = jax.lax.broadcasted_iota(jnp.int32, (1, 1, _LANE), 2)

    def step(i, carry):
        num = carry
        content = sem_ref[pl.ds(i, 1), :]
        ccol = content.reshape(_SEM_DIM, 1)
        ccol3 = ccol.reshape(1, _SEM_DIM, 1)

        def mac_scan(m, dcarry):
            dmin, jmin = dcarry
            tiles = tt_ref[pl.ds(m * _MAC, _MAC), :, :]
            diffs = tiles - ccol3
            d2 = jnp.sum(diffs * diffs, axis=1)
            ids = m * _MACSZ + seg_ids * _LANE + lane_ids
            d2m = jnp.where(ids < num, d2, _INF)
            bmin = jnp.min(d2m)
            bj = jnp.min(jnp.where(d2m == bmin, ids, _SLOTS))
            take = bmin < dmin
            return (jnp.where(take, bmin, dmin),
                    jnp.where(take, bj, jmin))

        nmac = (num + (_MACSZ - 1)) // _MACSZ
        dmin, j = jax.lax.fori_loop(
            0, nmac, mac_scan, (jnp.float32(_INF), jnp.int32(0)))
        do_update = (num > 0) & (dmin < 4.0)

        reward = jnp.abs(rewards_ref[pl.ds(i, 1), :][0, 0])
        eff_lr = _LR * (1.0 + reward)
        s_old = strengths_ref[pl.ds(j, 1), :]

        tgt = jnp.where(do_update, j, num)
        sg = tgt // _LANE
        lane_t = tgt - sg * _LANE
        tile = tt_ref[pl.ds(sg, 1), :, :]
        hit = lane_ids3 == lane_t
        old = jnp.sum(jnp.where(hit, tile, 0.0), axis=2)
        upd = old + (content - old) * eff_lr
        newrow = jnp.where(do_update, upd, content)
        newcol3 = newrow.reshape(1, _SEM_DIM, 1)
        tt_ref[pl.ds(sg, 1), :, :] = jnp.where(hit, newcol3, tile)
        strengths_ref[pl.ds(tgt, 1), :] = jnp.where(do_update, s_old + 1.0, 1.0)
        return jnp.where(do_update, num, num + 1)

    num = jax.lax.fori_loop(0, _N, step, jnp.int32(0))

    for sg in range(_SLOTS // _LANE):
        traces_ref[sg * _LANE:(sg + 1) * _LANE, :] = tt_ref[sg, :, :].T

    valid = row_ids < num
    total = jnp.sum(jnp.where(valid, strengths_ref[...], 0.0))
    ms = jnp.where(num > 0, total / jnp.maximum(num, 1).astype(jnp.float32), 0.0)
    num_ref[...] = jnp.full((1, 1), num, jnp.int32)
    ms_ref[...] = jnp.full((1, 1), ms, jnp.float32)


@jax.jit
def kernel(replayed_states, replayed_rewards, W, b):
    rewards2 = replayed_rewards.reshape(_N, 1)
    b2 = b.reshape(1, _SEM_DIM)
    traces, num, ms = pl.pallas_call(
        _consolidate_kernel,
        out_shape=[
            jax.ShapeDtypeStruct((_SLOTS, _SEM_DIM), jnp.float32),
            jax.ShapeDtypeStruct((1, 1), jnp.int32),
            jax.ShapeDtypeStruct((1, 1), jnp.float32),
        ],
        scratch_shapes=[
            pltpu.VMEM((_N, _SEM_DIM), jnp.float32),
            pltpu.VMEM((_SLOTS, 1), jnp.float32),
            pltpu.VMEM((_SLOTS // _LANE, _SEM_DIM, _LANE), jnp.float32),
        ],
    )(replayed_states, rewards2, W, b2)
    return (jnp.array(True), jnp.array(_N, jnp.int32), num[0, 0], ms[0, 0],
            traces)
